# baseline (speedup 1.0000x reference)
import jax
import jax.numpy as jnp
from jax.experimental import pallas as pl

_B, _S, _D, _H = 1, 2048, 1024, 16
_DH = _D // _H
_TOPK = 64
_SCALE = (_DH ** -0.5)
_QB = 1024


def _matmul_bias_kernel(x_ref, w_ref, b_ref, o_ref):
    o_ref[...] = (
        jnp.dot(x_ref[...], w_ref[...], preferred_element_type=jnp.float32)
        + b_ref[...]
    )


def _proj(x2d, w, b, block_rows=256):
    m, k = x2d.shape
    n = w.shape[1]
    return pl.pallas_call(
        _matmul_bias_kernel,
        grid=(m // block_rows,),
        in_specs=[
            pl.BlockSpec((block_rows, k), lambda i: (i, 0)),
            pl.BlockSpec((k, n), lambda i: (0, 0)),
            pl.BlockSpec((1, n), lambda i: (0, 0)),
        ],
        out_specs=pl.BlockSpec((block_rows, n), lambda i: (i, 0)),
        out_shape=jax.ShapeDtypeStruct((m, n), jnp.float32),
    )(x2d, w, b[None, :])


def _monotone_key(s):
    i = jax.lax.bitcast_convert_type(s, jnp.int32)
    int_min = jnp.int32(-2147483648)
    return jnp.where(i >= 0, i, int_min - i)


def _key_to_f32(k):
    int_min = jnp.int32(-2147483648)
    return jax.lax.bitcast_convert_type(jnp.where(k >= 0, k, int_min - k), jnp.float32)


def _attn_kernel(q_ref, k_ref, v_ref, o_ref):
    for h0 in (0, _DH):
        _attn_one_head(q_ref, k_ref, v_ref, o_ref, h0)


def _attn_one_head(q_ref, k_ref, v_ref, o_ref, h0):
    q = q_ref[:, h0:h0 + _DH] * _SCALE
    k = k_ref[:, h0:h0 + _DH]
    s = jax.lax.dot_general(
        q, k, (((1,), (1,)), ((), ())), preferred_element_type=jnp.float32
    )

    m = jnp.max(s, axis=1, keepdims=True)
    t = jnp.maximum(s[:, :1024], s[:, 1024:])
    t = jnp.maximum(t[:, :512], t[:, 512:])
    t = jnp.maximum(t[:, :256], t[:, 256:])
    cmax = jnp.maximum(t[:, :128], t[:, 128:])
    cmax = jnp.maximum(cmax[:, :64], cmax[:, 64:])
    m0 = jnp.min(cmax, axis=1, keepdims=True)
    lo0 = _monotone_key(m0)
    hi0 = _monotone_key(m) + 1

    def cond(carry):
        lo, hi = carry
        return jnp.any((hi - lo) > 1)

    def body(carry):
        lo, hi = carry
        mid = (lo >> 1) + (hi >> 1) + (lo & hi & 1)
        cnt = jnp.sum((s >= _key_to_f32(mid)).astype(jnp.int32), axis=1,
                      keepdims=True)
        ge = cnt >= _TOPK
        eq = cnt == _TOPK
        lo = jnp.where(ge, mid, lo)
        hi = jnp.where(eq, mid + 1, jnp.where(ge, hi, mid))
        return lo, hi

    lo, _ = jax.lax.while_loop(cond, body, (lo0, hi0))

    e = jnp.where(s >= _key_to_f32(lo), jnp.exp(s - m), 0.0)
    attn = e / jnp.sum(e, axis=1, keepdims=True)
    o_ref[:, h0:h0 + _DH] = jnp.dot(
        attn, v_ref[:, h0:h0 + _DH], preferred_element_type=jnp.float32
    )


def kernel(x, Wq, bq, Wk, bk, Wv, bv, Wo, bo):
    b, s, d = x.shape
    x2d = x.reshape(s, d)

    w_qkv = jnp.concatenate([Wq, Wk, Wv], axis=1)
    b_qkv = jnp.concatenate([bq, bk, bv], axis=0)
    qkv = _proj(x2d, w_qkv, b_qkv)

    ctx2d = pl.pallas_call(
        _attn_kernel,
        grid=(_H // 2, _S // _QB),
        in_specs=[
            pl.BlockSpec((_QB, 2 * _DH), lambda p, i: (i, p)),
            pl.BlockSpec((_S, 2 * _DH), lambda p, i: (0, 8 + p)),
            pl.BlockSpec((_S, 2 * _DH), lambda p, i: (0, 16 + p)),
        ],
        out_specs=pl.BlockSpec((_QB, 2 * _DH), lambda p, i: (i, p)),
        out_shape=jax.ShapeDtypeStruct((_S, _D), jnp.float32),
    )(qkv, qkv, qkv)

    out = _proj(ctx2d, Wo, bo)
    return out.reshape(b, s, d)


# speedup vs baseline: 1.0034x; 1.0034x over previous
import jax
import jax.numpy as jnp
from jax.experimental import pallas as pl

_B, _S, _D, _H = 1, 2048, 1024, 16
_DH = _D // _H
_TOPK = 64
_SCALE = (_DH ** -0.5)
_QB = 1024


def _matmul_bias_kernel(x_ref, w_ref, b_ref, o_ref):
    o_ref[...] = (
        jnp.dot(x_ref[...], w_ref[...], preferred_element_type=jnp.float32)
        + b_ref[...]
    )


def _proj(x2d, w, b, block_rows=256):
    m, k = x2d.shape
    n = w.shape[1]
    return pl.pallas_call(
        _matmul_bias_kernel,
        grid=(m // block_rows,),
        in_specs=[
            pl.BlockSpec((block_rows, k), lambda i: (i, 0)),
            pl.BlockSpec((k, n), lambda i: (0, 0)),
            pl.BlockSpec((1, n), lambda i: (0, 0)),
        ],
        out_specs=pl.BlockSpec((block_rows, n), lambda i: (i, 0)),
        out_shape=jax.ShapeDtypeStruct((m, n), jnp.float32),
    )(x2d, w, b[None, :])


def _monotone_key(s):
    i = jax.lax.bitcast_convert_type(s, jnp.int32)
    int_min = jnp.int32(-2147483648)
    return jnp.where(i >= 0, i, int_min - i)


def _key_to_f32(k):
    int_min = jnp.int32(-2147483648)
    return jax.lax.bitcast_convert_type(jnp.where(k >= 0, k, int_min - k), jnp.float32)


def _attn_kernel(q_ref, k_ref, v_ref, o_ref):
    for h0 in (0, _DH):
        _attn_one_head(q_ref, k_ref, v_ref, o_ref, h0)


def _attn_one_head(q_ref, k_ref, v_ref, o_ref, h0):
    q = q_ref[:, h0:h0 + _DH] * _SCALE
    k = k_ref[:, h0:h0 + _DH]
    s = jax.lax.dot_general(
        q, k, (((1,), (1,)), ((), ())), preferred_element_type=jnp.float32
    )

    m = jnp.max(s, axis=1, keepdims=True)
    t = jnp.maximum(s[:, :1024], s[:, 1024:])
    t = jnp.maximum(t[:, :512], t[:, 512:])
    t = jnp.maximum(t[:, :256], t[:, 256:])
    cmax = jnp.maximum(t[:, :128], t[:, 128:])
    m0 = jnp.min(cmax, axis=1, keepdims=True)
    lo0 = _monotone_key(m0)
    hi0 = _monotone_key(m) + 1

    def cond(carry):
        lo, hi = carry
        return jnp.any((hi - lo) > 1)

    def body(carry):
        lo, hi = carry
        mid = (lo >> 1) + (hi >> 1) + (lo & hi & 1)
        cnt = jnp.sum((s >= _key_to_f32(mid)).astype(jnp.int32), axis=1,
                      keepdims=True)
        ge = cnt >= _TOPK
        eq = cnt == _TOPK
        lo = jnp.where(ge, mid, lo)
        hi = jnp.where(eq, mid + 1, jnp.where(ge, hi, mid))
        return lo, hi

    lo, _ = jax.lax.while_loop(cond, body, (lo0, hi0))

    e = jnp.where(s >= _key_to_f32(lo), jnp.exp(s - m), 0.0)
    attn = e / jnp.sum(e, axis=1, keepdims=True)
    o_ref[:, h0:h0 + _DH] = jnp.dot(
        attn, v_ref[:, h0:h0 + _DH], preferred_element_type=jnp.float32
    )


def kernel(x, Wq, bq, Wk, bk, Wv, bv, Wo, bo):
    b, s, d = x.shape
    x2d = x.reshape(s, d)

    w_qkv = jnp.concatenate([Wq, Wk, Wv], axis=1)
    b_qkv = jnp.concatenate([bq, bk, bv], axis=0)
    qkv = _proj(x2d, w_qkv, b_qkv)

    ctx2d = pl.pallas_call(
        _attn_kernel,
        grid=(_H // 2, _S // _QB),
        in_specs=[
            pl.BlockSpec((_QB, 2 * _DH), lambda p, i: (i, p)),
            pl.BlockSpec((_S, 2 * _DH), lambda p, i: (0, 8 + p)),
            pl.BlockSpec((_S, 2 * _DH), lambda p, i: (0, 16 + p)),
        ],
        out_specs=pl.BlockSpec((_QB, 2 * _DH), lambda p, i: (i, p)),
        out_shape=jax.ShapeDtypeStruct((_S, _D), jnp.float32),
    )(qkv, qkv, qkv)

    out = _proj(ctx2d, Wo, bo)
    return out.reshape(b, s, d)
